# baseline (speedup 1.0000x reference)
import jax
import jax.numpy as jnp
from jax.experimental import pallas as pl
from jax.experimental.pallas import tpu as pltpu

B = 2048
H = 16
A = 32
IN = 512
NL = 1024
NE = 1024
NG = 8
CW = 0.25
CHUNK = 512


def _dot(a, b, trans_b=False):
    dims = (((1,), (1 if trans_b else 0,)), ((), ()))
    return jax.lax.dot_general(a, b, dims, preferred_element_type=jnp.float32)


def _enc_body(x_ref, w1_ref, b1_ref, w2_ref, b2_ref, w3_ref, b3_ref, z_ref):
    z = jnp.maximum(_dot(x_ref[...], w1_ref[...]) + b1_ref[...], 0.0)
    z = jnp.maximum(_dot(z, w2_ref[...]) + b2_ref[...], 0.0)
    z_ref[...] = _dot(z, w3_ref[...]) + b3_ref[...]


def _vq_body(z_ref, emb_ref, idx_ref, quant_ref, loss_ref,
             res_scr, e1_scr, e2_scr, e3_scr, en_scr):
    g = pl.program_id(0)
    c = pl.program_id(1)
    rows = pl.ds(c * CHUNK, CHUNK)

    @pl.when(jnp.logical_and(g == 0, c == 0))
    def _():
        loss_ref[...] = jnp.zeros_like(loss_ref)

    E = emb_ref[0]

    @pl.when(c == 0)
    def _():
        e1 = E.astype(jnp.bfloat16)
        r1 = E - e1.astype(jnp.float32)
        e2 = r1.astype(jnp.bfloat16)
        e1_scr[...] = e1
        e2_scr[...] = e2
        e3_scr[...] = (r1 - e2.astype(jnp.float32)).astype(jnp.bfloat16)
        en_scr[...] = jnp.sum(E * E, axis=1)[None, :]

    @pl.when(g == 0)
    def _():
        res_scr[rows, :] = z_ref[rows, :]
        quant_ref[rows, :] = jnp.zeros((CHUNK, NL), jnp.float32)

    r = res_scr[rows, :]
    s = _dot(r, E, trans_b=True)
    rn = jnp.sum(r * r, axis=1, keepdims=True)
    dist = (rn - 2.0 * s) + en_scr[...]
    m = jnp.min(dist, axis=1, keepdims=True)
    iota = jax.lax.broadcasted_iota(jnp.int32, dist.shape, 1)
    idx = jnp.min(jnp.where(dist == m, iota, NE), axis=1)
    oh = (iota == idx[:, None]).astype(jnp.bfloat16)
    q = (_dot(oh, e1_scr[...]) + _dot(oh, e2_scr[...])) + _dot(oh, e3_scr[...])
    q_st = r + (q - r)
    quant = quant_ref[rows, :] + q_st
    quant_ref[rows, :] = quant
    res_scr[rows, :] = z_ref[rows, :] - quant
    d = r - q
    loss_ref[...] += (jnp.sum(d * d) * ((1.0 + CW) / (B * NL))).reshape(1, 1)
    idx_ref[...] = idx.reshape(1, 1, CHUNK)


def _dec_body(q_ref, x_ref, w1_ref, b1_ref, w2_ref, b2_ref, w3_ref, b3_ref,
              rec_ref, loss_ref):
    c = pl.program_id(0)

    @pl.when(c == 0)
    def _():
        loss_ref[...] = jnp.zeros_like(loss_ref)

    h = jnp.maximum(_dot(q_ref[...], w1_ref[...]) + b1_ref[...], 0.0)
    h = jnp.maximum(_dot(h, w2_ref[...]) + b2_ref[...], 0.0)
    rec = _dot(h, w3_ref[...]) + b3_ref[...]
    rec_ref[...] = rec
    d = rec - x_ref[...]
    loss_ref[...] += (jnp.sum(d * d) * (1.0 / (B * IN))).reshape(1, 1)


def _full(shape):
    return pl.BlockSpec(shape, lambda *_: tuple(0 for _ in shape))


def kernel(actions, enc_W1, enc_b1, enc_W2, enc_b2, enc_W3, enc_b3,
           dec_W1, dec_b1, dec_W2, dec_b2, dec_W3, dec_b3, embed):
    x = actions.reshape(B, IN)
    nc = B // CHUNK

    z = pl.pallas_call(
        _enc_body,
        grid=(nc,),
        in_specs=[
            pl.BlockSpec((CHUNK, IN), lambda c: (c, 0)),
            _full((IN, NL)), _full((1, NL)),
            _full((NL, NL)), _full((1, NL)),
            _full((NL, NL)), _full((1, NL)),
        ],
        out_specs=pl.BlockSpec((CHUNK, NL), lambda c: (c, 0)),
        out_shape=jax.ShapeDtypeStruct((B, NL), jnp.float32),
    )(x, enc_W1, enc_b1.reshape(1, NL), enc_W2, enc_b2.reshape(1, NL),
      enc_W3, enc_b3.reshape(1, NL))

    idx, quant, vq_loss = pl.pallas_call(
        _vq_body,
        grid=(NG, nc),
        in_specs=[
            pl.BlockSpec((B, NL), lambda g, c: (0, 0)),
            pl.BlockSpec((1, NE, NL), lambda g, c: (g, 0, 0)),
        ],
        out_specs=[
            pl.BlockSpec((1, 1, CHUNK), lambda g, c: (g, 0, c)),
            pl.BlockSpec((B, NL), lambda g, c: (0, 0)),
            pl.BlockSpec((1, 1), lambda g, c: (0, 0)),
        ],
        out_shape=[
            jax.ShapeDtypeStruct((NG, 1, B), jnp.int32),
            jax.ShapeDtypeStruct((B, NL), jnp.float32),
            jax.ShapeDtypeStruct((1, 1), jnp.float32),
        ],
        scratch_shapes=[
            pltpu.VMEM((B, NL), jnp.float32),
            pltpu.VMEM((NE, NL), jnp.bfloat16),
            pltpu.VMEM((NE, NL), jnp.bfloat16),
            pltpu.VMEM((NE, NL), jnp.bfloat16),
            pltpu.VMEM((1, NE), jnp.float32),
        ],
    )(z, embed)

    rec, rec_loss = pl.pallas_call(
        _dec_body,
        grid=(nc,),
        in_specs=[
            pl.BlockSpec((CHUNK, NL), lambda c: (c, 0)),
            pl.BlockSpec((CHUNK, IN), lambda c: (c, 0)),
            _full((NL, NL)), _full((1, NL)),
            _full((NL, NL)), _full((1, NL)),
            _full((NL, IN)), _full((1, IN)),
        ],
        out_specs=[
            pl.BlockSpec((CHUNK, IN), lambda c: (c, 0)),
            pl.BlockSpec((1, 1), lambda c: (0, 0)),
        ],
        out_shape=[
            jax.ShapeDtypeStruct((B, IN), jnp.float32),
            jax.ShapeDtypeStruct((1, 1), jnp.float32),
        ],
    )(quant, x, dec_W1, dec_b1.reshape(1, NL), dec_W2, dec_b2.reshape(1, NL),
      dec_W3, dec_b3.reshape(1, IN))

    reconstructed = rec.reshape(B, H, A)
    indices = idx.reshape(NG, B).T
    total_loss = vq_loss[0, 0] + rec_loss[0, 0]
    return (reconstructed, indices, total_loss)


# speedup vs baseline: 1.0302x; 1.0302x over previous
import jax
import jax.numpy as jnp
from jax.experimental import pallas as pl
from jax.experimental.pallas import tpu as pltpu

B = 2048
H = 16
A = 32
IN = 512
NL = 1024
NE = 1024
NG = 8
CW = 0.25
CHUNK = 512


def _dot(a, b, trans_b=False):
    dims = (((1,), (1 if trans_b else 0,)), ((), ()))
    return jax.lax.dot_general(a, b, dims, preferred_element_type=jnp.float32)


def _enc_body(x_ref, w1_ref, b1_ref, w2_ref, b2_ref, w3_ref, b3_ref, z_ref):
    z = jnp.maximum(_dot(x_ref[...], w1_ref[...]) + b1_ref[...], 0.0)
    z = jnp.maximum(_dot(z, w2_ref[...]) + b2_ref[...], 0.0)
    z_ref[...] = _dot(z, w3_ref[...]) + b3_ref[...]


def _vq_body(z_ref, emb_ref, x_ref, w1_ref, b1_ref, w2_ref, b2_ref,
             w3_ref, b3_ref, idx_ref, rec_ref, loss_ref, rloss_ref,
             quant_scr, e1_scr, e2_scr, e3_scr, en_scr):
    g = pl.program_id(0)
    c = pl.program_id(1)
    rows = pl.ds(c * CHUNK, CHUNK)

    @pl.when(jnp.logical_and(g == 0, c == 0))
    def _():
        loss_ref[...] = jnp.zeros_like(loss_ref)
        rloss_ref[...] = jnp.zeros_like(rloss_ref)

    E = emb_ref[0]

    @pl.when(c == 0)
    def _():
        e1 = E.astype(jnp.bfloat16)
        r1 = E - e1.astype(jnp.float32)
        e2 = r1.astype(jnp.bfloat16)
        e1_scr[...] = e1
        e2_scr[...] = e2
        e3_scr[...] = (r1 - e2.astype(jnp.float32)).astype(jnp.bfloat16)
        en_scr[...] = jnp.sum(E * E, axis=1)[None, :]

    @pl.when(g == 0)
    def _():
        quant_scr[rows, :] = jnp.zeros((CHUNK, NL), jnp.float32)

    r = z_ref[rows, :] - quant_scr[rows, :]
    s = _dot(r, E, trans_b=True)
    rn = jnp.sum(r * r, axis=1, keepdims=True)
    dist = (rn - 2.0 * s) + en_scr[...]
    m = jnp.min(dist, axis=1, keepdims=True)
    iota = jax.lax.broadcasted_iota(jnp.int32, dist.shape, 1)
    idx = jnp.min(jnp.where(dist == m, iota, NE), axis=1)
    oh = (iota == idx[:, None]).astype(jnp.bfloat16)
    q = (_dot(oh, e1_scr[...]) + _dot(oh, e2_scr[...])) + _dot(oh, e3_scr[...])
    q_st = r + (q - r)
    quant = quant_scr[rows, :] + q_st
    quant_scr[rows, :] = quant
    d = r - q
    loss_ref[...] += (jnp.sum(d * d) * ((1.0 + CW) / (B * NL))).reshape(1, 1)
    idx_ref[...] = idx.reshape(1, 1, CHUNK)

    @pl.when(g == NG - 1)
    def _():
        h = jnp.maximum(_dot(quant, w1_ref[...]) + b1_ref[...], 0.0)
        h = jnp.maximum(_dot(h, w2_ref[...]) + b2_ref[...], 0.0)
        rec = _dot(h, w3_ref[...]) + b3_ref[...]
        rec_ref[...] = rec
        dr = rec - x_ref[rows, :]
        rloss_ref[...] += (jnp.sum(dr * dr) * (1.0 / (B * IN))).reshape(1, 1)


def _full(shape):
    return pl.BlockSpec(shape, lambda *_: tuple(0 for _ in shape))


def kernel(actions, enc_W1, enc_b1, enc_W2, enc_b2, enc_W3, enc_b3,
           dec_W1, dec_b1, dec_W2, dec_b2, dec_W3, dec_b3, embed):
    x = actions.reshape(B, IN)
    nc = B // CHUNK

    z = pl.pallas_call(
        _enc_body,
        grid=(nc,),
        in_specs=[
            pl.BlockSpec((CHUNK, IN), lambda c: (c, 0)),
            _full((IN, NL)), _full((1, NL)),
            _full((NL, NL)), _full((1, NL)),
            _full((NL, NL)), _full((1, NL)),
        ],
        out_specs=pl.BlockSpec((CHUNK, NL), lambda c: (c, 0)),
        out_shape=jax.ShapeDtypeStruct((B, NL), jnp.float32),
    )(x, enc_W1, enc_b1.reshape(1, NL), enc_W2, enc_b2.reshape(1, NL),
      enc_W3, enc_b3.reshape(1, NL))

    idx, rec, vq_loss, rec_loss = pl.pallas_call(
        _vq_body,
        grid=(NG, nc),
        in_specs=[
            pl.BlockSpec((B, NL), lambda g, c: (0, 0)),
            pl.BlockSpec((1, NE, NL), lambda g, c: (g, 0, 0)),
            pl.BlockSpec((B, IN), lambda g, c: (0, 0)),
            _full((NL, NL)), _full((1, NL)),
            _full((NL, NL)), _full((1, NL)),
            _full((NL, IN)), _full((1, IN)),
        ],
        out_specs=[
            pl.BlockSpec((1, 1, CHUNK), lambda g, c: (g, 0, c)),
            pl.BlockSpec((CHUNK, IN), lambda g, c: (c, 0)),
            pl.BlockSpec((1, 1), lambda g, c: (0, 0)),
            pl.BlockSpec((1, 1), lambda g, c: (0, 0)),
        ],
        out_shape=[
            jax.ShapeDtypeStruct((NG, 1, B), jnp.int32),
            jax.ShapeDtypeStruct((B, IN), jnp.float32),
            jax.ShapeDtypeStruct((1, 1), jnp.float32),
            jax.ShapeDtypeStruct((1, 1), jnp.float32),
        ],
        scratch_shapes=[
            pltpu.VMEM((B, NL), jnp.float32),
            pltpu.VMEM((NE, NL), jnp.bfloat16),
            pltpu.VMEM((NE, NL), jnp.bfloat16),
            pltpu.VMEM((NE, NL), jnp.bfloat16),
            pltpu.VMEM((1, NE), jnp.float32),
        ],
    )(z, embed, x, dec_W1, dec_b1.reshape(1, NL), dec_W2,
      dec_b2.reshape(1, NL), dec_W3, dec_b3.reshape(1, IN))

    reconstructed = rec.reshape(B, H, A)
    indices = idx.reshape(NG, B).T
    total_loss = vq_loss[0, 0] + rec_loss[0, 0]
    return (reconstructed, indices, total_loss)
